# baseline (speedup 1.0000x reference)
import functools

import jax
import jax.numpy as jnp
from jax.experimental import pallas as pl
from jax.experimental.pallas import tpu as pltpu

N = 100000
BLK = 2000


def _tail_body(h1_ref, neigh_ref, wself_ref, wneigh_ref, bneigh_ref,
               wlin_ref, blin_ref, out_ref):
    i = pl.program_id(0)
    h1 = h1_ref[...]
    neigh = neigh_ref[...]
    h2 = jnp.maximum(
        h1 @ wself_ref[...] + neigh @ wneigh_ref[...] + bneigh_ref[...], 0.0)
    h3 = jnp.maximum(h2 @ wlin_ref[...] + blin_ref[...], 0.0)
    part = jnp.sum(h3, axis=0, keepdims=True)

    @pl.when(i == 0)
    def _():
        out_ref[...] = jnp.zeros_like(out_ref)

    out_ref[0:1, :] += part


def _dense_tail(h1, neigh, Wself, Wneigh, bneigh, Wlin, blin):
    grid = (N // BLK,)
    return pl.pallas_call(
        _tail_body,
        grid=grid,
        in_specs=[
            pl.BlockSpec((BLK, 32), lambda i: (i, 0)),
            pl.BlockSpec((BLK, 32), lambda i: (i, 0)),
            pl.BlockSpec((32, 64), lambda i: (0, 0)),
            pl.BlockSpec((32, 64), lambda i: (0, 0)),
            pl.BlockSpec((64,), lambda i: (0,)),
            pl.BlockSpec((64, 64), lambda i: (0, 0)),
            pl.BlockSpec((64,), lambda i: (0,)),
        ],
        out_specs=pl.BlockSpec((8, 64), lambda i: (0, 0)),
        out_shape=jax.ShapeDtypeStruct((8, 64), jnp.float32),
    )(h1, neigh, Wself, Wneigh, bneigh, Wlin, blin)


def kernel(node_tokens, edge_index, embed, W1, b1, Wpool, bpool, Wself,
           Wneigh, bneigh, Wlin, blin):
    src = edge_index[0]
    dst = edge_index[1]
    h = jnp.take(embed, node_tokens, axis=0)
    ones = jnp.ones((src.shape[0],), dtype=jnp.float32)
    out_deg = jax.ops.segment_sum(ones, src, num_segments=N)
    in_deg = jax.ops.segment_sum(ones, dst, num_segments=N)
    c_src = jax.lax.rsqrt(jnp.clip(out_deg, 1.0))
    c_dst = jax.lax.rsqrt(jnp.clip(in_deg, 1.0))
    hn = h * c_src[:, None]
    agg = jax.ops.segment_sum(jnp.take(hn, src, axis=0), dst, num_segments=N)
    h1 = jax.nn.relu((agg * c_dst[:, None]) @ W1 + b1)
    hp = jax.nn.relu(h1 @ Wpool + bpool)
    neigh = jax.ops.segment_max(jnp.take(hp, src, axis=0), dst, num_segments=N)
    neigh = jnp.where(in_deg[:, None] > 0, neigh, 0.0)
    part = _dense_tail(h1, neigh, Wself, Wneigh, bneigh, Wlin, blin)
    return jnp.sum(part, axis=0, keepdims=True)


# speedup vs baseline: 5.8238x; 5.8238x over previous
import functools

import jax
import jax.numpy as jnp
from jax import lax
from jax.experimental import pallas as pl
from jax.experimental.pallas import tpu as pltpu
from jax.experimental.pallas import tpu_sc as plsc

N = 100000
E = 3200000
VOCAB = 100000
NP = 102400
EP = 3276800
NW = 32
RPW = NP // NW
BIN = RPW
ECH = 2048
ECH_ROWS = ECH // 128
EROWS_PW = EP // NW // 128
NCHUNK_E = EP // NW // ECH
CAP = 8192
STG = 1024
FLUSH = 512
CCH = 512

_MESH = functools.partial(plsc.VectorSubcoreMesh,
                          core_axis_name="c", subcore_axis_name="s")
_SC_PARAMS = pltpu.CompilerParams(use_tc_tiling_on_sc=False,
                                  needs_layout_passes=False)


def _wid():
    return lax.axis_index("s") * 2 + lax.axis_index("c")


def _iota16():
    return lax.iota(jnp.int32, 16)


def _mo(x, n):
    return pl.multiple_of(x, n)


def _embed_gather_body(tok_hbm, emb_hbm, out_hbm, idx_v, rows_v, sem):
    w = _wid()
    for j in range(RPW // 128):
        pltpu.sync_copy(tok_hbm.at[pl.ds(_mo(w * RPW + j * 128, 128), 128)], idx_v.at[j])
    for j in range(RPW // 128):
        pltpu.async_copy(emb_hbm.at[idx_v.at[j]], rows_v, sem).wait()
        pltpu.sync_copy(rows_v, out_hbm.at[pl.ds(_mo(w * RPW + j * 128, 128), 128)])


def _sc_embed_gather(tokens_pad, embed):
    k = pl.kernel(
        _embed_gather_body,
        out_type=jax.ShapeDtypeStruct((NP, 16), jnp.float32),
        mesh=_MESH(),
        compiler_params=_SC_PARAMS,
        scratch_types=[
            pltpu.VMEM((RPW // 128, 128), jnp.int32),
            pltpu.VMEM((128, 16), jnp.float32),
            pltpu.SemaphoreType.DMA,
        ],
    )
    return k(tokens_pad, embed)


def _bin_degree_body(ef_hbm, hist_hbm, bsrc_hbm, bdst_hbm, cnt_hbm,
                     src_v, dst_v, src2, dst2, stage_s, stage_d,
                     cursor_v, counts_v, ones_v, zb_v, sh_src, sh_dst,
                     hcur, sem):
    cid = lax.axis_index("c")
    sid = lax.axis_index("s")
    w = _wid()
    zeros16i = jnp.zeros((16,), jnp.int32)
    for i in range(8):
        ones_v[pl.ds(i * 16, 16)] = jnp.ones((16,), jnp.float32)
    for i in range(2):
        cursor_v[pl.ds(i * 16, 16)] = zeros16i
    for b in range(32):
        hcur[b] = 0

    def zero_body(i, _):
        zb_v[pl.ds(_mo(i * 16, 16), 16)] = jnp.zeros((16,), jnp.float32)
        return 0

    lax.fori_loop(0, (NP // 16) // 16, zero_body, 0)
    pltpu.sync_copy(zb_v, sh_src.at[pl.ds(_mo(sid * (NP // 16), 64), NP // 16)])
    pltpu.sync_copy(zb_v, sh_dst.at[pl.ds(_mo(sid * (NP // 16), 64), NP // 16)])
    plsc.subcore_barrier()

    def _cursor(b):
        return cursor_v[pl.ds(16 * (b // 16), 16)][b % 16]

    def flush_scan():
        for b in range(32):
            cur = _cursor(b)

            @pl.when(cur >= FLUSH)
            def _():
                hb = hcur[b]
                pltpu.sync_copy(
                    stage_s.at[pl.ds(b * STG, FLUSH)],
                    bsrc_hbm.at[pl.ds(_mo((w * 32 + b) * CAP + hb, 512), FLUSH)])
                pltpu.sync_copy(
                    stage_d.at[pl.ds(b * STG, FLUSH)],
                    bdst_hbm.at[pl.ds(_mo((w * 32 + b) * CAP + hb, 512), FLUSH)])
                hcur[b] = jnp.minimum(hb + FLUSH, CAP - FLUSH)
                for k in range(8):
                    v1 = stage_s[pl.ds(b * STG + FLUSH + k * 16, 16)]
                    stage_s[pl.ds(b * STG + k * 16, 16)] = v1
                    v2 = stage_d[pl.ds(b * STG + FLUSH + k * 16, 16)]
                    stage_d[pl.ds(b * STG + k * 16, 16)] = v2
                plsc.store_scatter(
                    cursor_v, [jnp.full((16,), b, jnp.int32)],
                    jnp.full((16,), cur - FLUSH, jnp.int32),
                    mask=_iota16() == 0)

    def vreg_body(v, _):
        sv = src2[pl.ds(_mo(v * 16, 16), 16)]
        dv = dst2[pl.ds(_mo(v * 16, 16), 16)]
        binv = dv // BIN
        dloc = dv - binv * BIN
        occ, lastm = plsc.scan_count(binv)
        base = plsc.load_gather(cursor_v, [binv])
        pos = base + occ - 1
        fidx = binv * STG + pos
        plsc.store_scatter(stage_s, [fidx], sv)
        plsc.store_scatter(stage_d, [fidx], dloc)
        plsc.store_scatter(cursor_v, [binv], base + occ, mask=lastm)

        @pl.when((v & 7) == 7)
        def _():
            c1 = jnp.max(cursor_v[pl.ds(0, 16)])
            c2 = jnp.max(cursor_v[pl.ds(16, 16)])

            @pl.when(jnp.maximum(c1, c2) >= FLUSH)
            def _():
                flush_scan()
        return 0

    def chunk_body(j, _):
        ebase = w * (EP // NW) + j * ECH
        pltpu.sync_copy(ef_hbm.at[0, pl.ds(_mo(ebase, 2048), ECH)], src2)
        pltpu.sync_copy(ef_hbm.at[1, pl.ds(_mo(ebase, 2048), ECH)], dst2)
        for r in range(ECH_ROWS):
            pltpu.sync_copy(
                ef_hbm.at[0, pl.ds(_mo(ebase + r * 128, 128), 128)],
                src_v.at[r])
            pltpu.sync_copy(
                ef_hbm.at[1, pl.ds(_mo(ebase + r * 128, 128), 128)],
                dst_v.at[r])
        descs = []
        for r in range(ECH_ROWS):
            descs.append(pltpu.async_copy(
                ones_v, sh_src.at[src_v.at[r]], sem, add=True))
            descs.append(pltpu.async_copy(
                ones_v, sh_dst.at[dst_v.at[r]], sem, add=True))
        lax.fori_loop(0, ECH // 16, vreg_body, 0)
        for d in descs:
            d.wait()
        return 0

    lax.fori_loop(0, NCHUNK_E, chunk_body, 0)

    for b in range(32):
        cur = _cursor(b)
        hb = hcur[b]
        pltpu.sync_copy(
            stage_s.at[pl.ds(b * STG, FLUSH)],
            bsrc_hbm.at[pl.ds(_mo((w * 32 + b) * CAP + hb, 512), FLUSH)])
        pltpu.sync_copy(
            stage_d.at[pl.ds(b * STG, FLUSH)],
            bdst_hbm.at[pl.ds(_mo((w * 32 + b) * CAP + hb, 512), FLUSH)])
        plsc.store_scatter(
            counts_v, [jnp.full((16,), b, jnp.int32)],
            jnp.full((16,), jnp.minimum(hb + cur, CAP), jnp.int32),
            mask=_iota16() == 0)
    pltpu.sync_copy(counts_v, cnt_hbm.at[pl.ds(_mo(w * 32, 32), 32)])

    plsc.subcore_barrier()

    @pl.when(sid == 0)
    def _():
        pltpu.sync_copy(sh_src, hist_hbm.at[cid, 0])
        pltpu.sync_copy(sh_dst, hist_hbm.at[cid, 1])


def _sc_bin_degrees(edges_flat):
    k = pl.kernel(
        _bin_degree_body,
        out_type=[
            jax.ShapeDtypeStruct((2, 2, NP), jnp.float32),
            jax.ShapeDtypeStruct((NW * 32 * CAP,), jnp.int32),
            jax.ShapeDtypeStruct((NW * 32 * CAP,), jnp.int32),
            jax.ShapeDtypeStruct((NW * 32,), jnp.int32),
        ],
        mesh=_MESH(),
        compiler_params=_SC_PARAMS,
        scratch_types=[
            pltpu.VMEM((ECH_ROWS, 128), jnp.int32),
            pltpu.VMEM((ECH_ROWS, 128), jnp.int32),
            pltpu.VMEM((ECH,), jnp.int32),
            pltpu.VMEM((ECH,), jnp.int32),
            pltpu.VMEM((32 * STG,), jnp.int32),
            pltpu.VMEM((32 * STG,), jnp.int32),
            pltpu.VMEM((32,), jnp.int32),
            pltpu.VMEM((32,), jnp.int32),
            pltpu.VMEM((128,), jnp.float32),
            pltpu.VMEM((NP // 16,), jnp.float32),
            pltpu.VMEM_SHARED((NP,), jnp.float32),
            pltpu.VMEM_SHARED((NP,), jnp.float32),
            pltpu.SMEM((32,), jnp.int32),
            pltpu.SemaphoreType.DMA,
        ],
    )
    return k(edges_flat)


def _seg_sum_body(bsrc_hbm, bdst_hbm, cnt_hbm, hn_hbm, agg_hbm,
                  esrc_v, edst_v, rows_v, acc_f, cnt_v, sem):
    t = _wid()
    iota = _iota16()
    pltpu.sync_copy(cnt_hbm, cnt_v)

    def zacc(i, _):
        acc_f[pl.ds(_mo(i * 16, 16), 16)] = jnp.zeros((16,), jnp.float32)
        return 0

    lax.fori_loop(0, BIN, zacc, 0)

    def edge_body(e, _):
        e16 = jnp.full((16,), e, jnp.int32)
        d = plsc.load_gather(edst_v, [e16])[0]
        row = plsc.load_gather(rows_v, [e16, iota])
        a = acc_f[pl.ds(_mo(d * 16, 16), 16)]
        acc_f[pl.ds(_mo(d * 16, 16), 16)] = a + row
        return 0

    def prod_body(p, _):
        cnt = plsc.load_gather(
            cnt_v, [jnp.full((16,), p * 32 + t, jnp.int32)])[0]
        nch = (cnt + CCH - 1) // CCH

        def cbody(c, _):
            off = (p * 32 + t) * CAP + c * CCH
            pltpu.sync_copy(bsrc_hbm.at[pl.ds(_mo(off, 512), CCH)], esrc_v)
            pltpu.sync_copy(bdst_hbm.at[pl.ds(_mo(off, 512), CCH)], edst_v)

            def clampb(k, _):
                x = esrc_v[pl.ds(_mo(k * 16, 16), 16)]
                esrc_v[pl.ds(_mo(k * 16, 16), 16)] = jnp.clip(x, 0, NP - 1)
                return 0

            lax.fori_loop(0, CCH // 16, clampb, 0)
            for k in range(CCH // 128):
                pltpu.async_copy(
                    hn_hbm.at[esrc_v.at[pl.ds(k * 128, 128)]],
                    rows_v.at[pl.ds(k * 128, 128)], sem).wait()
            lax.fori_loop(0, jnp.minimum(CCH, cnt - c * CCH), edge_body, 0)
            return 0

        lax.fori_loop(0, nch, cbody, 0)
        return 0

    lax.fori_loop(0, 32, prod_body, 0)

    pltpu.sync_copy(acc_f, agg_hbm.at[pl.ds(_mo(t * BIN * 16, 512), BIN * 16)])


def _sc_seg_sum(bsrc, bdst, cnts, hn_pad):
    k = pl.kernel(
        _seg_sum_body,
        out_type=jax.ShapeDtypeStruct((NP * 16,), jnp.float32),
        mesh=_MESH(),
        compiler_params=_SC_PARAMS,
        scratch_types=[
            pltpu.VMEM((CCH,), jnp.int32),
            pltpu.VMEM((CCH,), jnp.int32),
            pltpu.VMEM((CCH, 16), jnp.float32),
            pltpu.VMEM((BIN * 16,), jnp.float32),
            pltpu.VMEM((NW * 32,), jnp.int32),
            pltpu.SemaphoreType.DMA,
        ],
    )
    return k(bsrc, bdst, cnts, hn_pad)


def _seg_max_body(bsrc_hbm, bdst_hbm, cnt_hbm, hpa_hbm, hpb_hbm,
                  nga_hbm, ngb_hbm,
                  esrc_v, edst_v, rowsa_v, rowsb_v, acca_f, accb_f, cnt_v,
                  sem):
    t = _wid()
    iota = _iota16()
    pltpu.sync_copy(cnt_hbm, cnt_v)

    def zacc(i, _):
        acca_f[pl.ds(_mo(i * 16, 16), 16)] = jnp.zeros((16,), jnp.float32)
        accb_f[pl.ds(_mo(i * 16, 16), 16)] = jnp.zeros((16,), jnp.float32)
        return 0

    lax.fori_loop(0, BIN, zacc, 0)

    def edge_body(e, _):
        e16 = jnp.full((16,), e, jnp.int32)
        d = plsc.load_gather(edst_v, [e16])[0]
        ra = plsc.load_gather(rowsa_v, [e16, iota])
        rb = plsc.load_gather(rowsb_v, [e16, iota])
        a = acca_f[pl.ds(_mo(d * 16, 16), 16)]
        acca_f[pl.ds(_mo(d * 16, 16), 16)] = jnp.maximum(a, ra)
        b = accb_f[pl.ds(_mo(d * 16, 16), 16)]
        accb_f[pl.ds(_mo(d * 16, 16), 16)] = jnp.maximum(b, rb)
        return 0

    def prod_body(p, _):
        cnt = plsc.load_gather(
            cnt_v, [jnp.full((16,), p * 32 + t, jnp.int32)])[0]
        nch = (cnt + CCH - 1) // CCH

        def cbody(c, _):
            off = (p * 32 + t) * CAP + c * CCH
            pltpu.sync_copy(bsrc_hbm.at[pl.ds(_mo(off, 512), CCH)], esrc_v)
            pltpu.sync_copy(bdst_hbm.at[pl.ds(_mo(off, 512), CCH)], edst_v)

            def clampb(k, _):
                x = esrc_v[pl.ds(_mo(k * 16, 16), 16)]
                esrc_v[pl.ds(_mo(k * 16, 16), 16)] = jnp.clip(x, 0, NP - 1)
                return 0

            lax.fori_loop(0, CCH // 16, clampb, 0)
            for k in range(CCH // 128):
                pltpu.async_copy(
                    hpa_hbm.at[esrc_v.at[pl.ds(k * 128, 128)]],
                    rowsa_v.at[pl.ds(k * 128, 128)], sem).wait()
                pltpu.async_copy(
                    hpb_hbm.at[esrc_v.at[pl.ds(k * 128, 128)]],
                    rowsb_v.at[pl.ds(k * 128, 128)], sem).wait()
            lax.fori_loop(0, jnp.minimum(CCH, cnt - c * CCH), edge_body, 0)
            return 0

        lax.fori_loop(0, nch, cbody, 0)
        return 0

    lax.fori_loop(0, 32, prod_body, 0)

    pltpu.sync_copy(acca_f, nga_hbm.at[pl.ds(_mo(t * BIN * 16, 512), BIN * 16)])
    pltpu.sync_copy(accb_f, ngb_hbm.at[pl.ds(_mo(t * BIN * 16, 512), BIN * 16)])


def _sc_seg_max(bsrc, bdst, cnts, hpa_pad, hpb_pad):
    k = pl.kernel(
        _seg_max_body,
        out_type=[
            jax.ShapeDtypeStruct((NP * 16,), jnp.float32),
            jax.ShapeDtypeStruct((NP * 16,), jnp.float32),
        ],
        mesh=_MESH(),
        compiler_params=_SC_PARAMS,
        scratch_types=[
            pltpu.VMEM((CCH,), jnp.int32),
            pltpu.VMEM((CCH,), jnp.int32),
            pltpu.VMEM((CCH, 16), jnp.float32),
            pltpu.VMEM((CCH, 16), jnp.float32),
            pltpu.VMEM((BIN * 16,), jnp.float32),
            pltpu.VMEM((BIN * 16,), jnp.float32),
            pltpu.VMEM((NW * 32,), jnp.int32),
            pltpu.SemaphoreType.DMA,
        ],
    )
    return k(bsrc, bdst, cnts, hpa_pad, hpb_pad)


BLK = 2000


def _tail_body(h1_ref, neigh_ref, wself_ref, wneigh_ref, bneigh_ref,
               wlin_ref, blin_ref, out_ref):
    i = pl.program_id(0)
    h1 = h1_ref[...]
    neigh = neigh_ref[...]
    h2 = jnp.maximum(
        h1 @ wself_ref[...] + neigh @ wneigh_ref[...] + bneigh_ref[...], 0.0)
    h3 = jnp.maximum(h2 @ wlin_ref[...] + blin_ref[...], 0.0)
    part = jnp.sum(h3, axis=0, keepdims=True)

    @pl.when(i == 0)
    def _():
        out_ref[...] = jnp.zeros_like(out_ref)

    out_ref[0:1, :] += part


def _dense_tail(h1, neigh, Wself, Wneigh, bneigh, Wlin, blin):
    return pl.pallas_call(
        _tail_body,
        grid=(N // BLK,),
        in_specs=[
            pl.BlockSpec((BLK, 32), lambda i: (i, 0)),
            pl.BlockSpec((BLK, 32), lambda i: (i, 0)),
            pl.BlockSpec((32, 64), lambda i: (0, 0)),
            pl.BlockSpec((32, 64), lambda i: (0, 0)),
            pl.BlockSpec((64,), lambda i: (0,)),
            pl.BlockSpec((64, 64), lambda i: (0, 0)),
            pl.BlockSpec((64,), lambda i: (0,)),
        ],
        out_specs=pl.BlockSpec((8, 64), lambda i: (0, 0)),
        out_shape=jax.ShapeDtypeStruct((8, 64), jnp.float32),
    )(h1, neigh, Wself, Wneigh, bneigh, Wlin, blin)


def kernel(node_tokens, edge_index, embed, W1, b1, Wpool, bpool, Wself,
           Wneigh, bneigh, Wlin, blin):
    tokens_pad = jnp.pad(node_tokens.astype(jnp.int32), (0, NP - N))
    edges_flat = jnp.pad(edge_index.astype(jnp.int32), ((0, 0), (0, EP - E)),
                         constant_values=N)
    h_pad = _sc_embed_gather(tokens_pad, embed)
    hist, bsrc, bdst, cnts = _sc_bin_degrees(edges_flat)
    deg = hist[0] + hist[1]
    out_deg = deg[0, :N]
    in_deg = deg[1, :N]

    c_src_pad = jax.lax.rsqrt(jnp.clip(deg[0], 1.0))
    c_dst = jax.lax.rsqrt(jnp.clip(in_deg, 1.0))
    hn_pad = h_pad * c_src_pad[:, None]

    agg = _sc_seg_sum(bsrc, bdst, cnts, hn_pad).reshape(NP, 16)[:N]
    h1 = jax.nn.relu((agg * c_dst[:, None]) @ W1 + b1)
    hp = jax.nn.relu(h1 @ Wpool + bpool)

    hp_pad = jnp.pad(hp, ((0, NP - N), (0, 0)))
    nga, ngb = _sc_seg_max(bsrc, bdst, cnts, hp_pad[:, :16].copy(),
                           hp_pad[:, 16:].copy())
    neigh = jnp.concatenate(
        [nga.reshape(NP, 16)[:N], ngb.reshape(NP, 16)[:N]], axis=1)

    part = _dense_tail(h1, neigh, Wself, Wneigh, bneigh, Wlin, blin)
    return jnp.sum(part, axis=0, keepdims=True)


# speedup vs baseline: 9.0210x; 1.5490x over previous
import functools

import jax
import jax.numpy as jnp
from jax import lax
from jax.experimental import pallas as pl
from jax.experimental.pallas import tpu as pltpu
from jax.experimental.pallas import tpu_sc as plsc

N = 100000
E = 3200000
VOCAB = 100000
NP = 102400
EP = 3276800
NW = 32
RPW = NP // NW
BIN = RPW
ECH = 2048
ECH_ROWS = ECH // 128
EROWS_PW = EP // NW // 128
NCHUNK_E = EP // NW // ECH
CAP = 8192
STG = 1024
FLUSH = 512
CCH = 512

_MESH = functools.partial(plsc.VectorSubcoreMesh,
                          core_axis_name="c", subcore_axis_name="s")
_SC_PARAMS = pltpu.CompilerParams(use_tc_tiling_on_sc=False,
                                  needs_layout_passes=False)


def _wid():
    return lax.axis_index("s") * 2 + lax.axis_index("c")


def _iota16():
    return lax.iota(jnp.int32, 16)


def _mo(x, n):
    return pl.multiple_of(x, n)


def _embed_gather_body(tok_hbm, emb_hbm, out_hbm, idx_v, rows_v, sem):
    w = _wid()
    for j in range(RPW // 128):
        pltpu.sync_copy(tok_hbm.at[pl.ds(_mo(w * RPW + j * 128, 128), 128)], idx_v.at[j])
    for j in range(RPW // 128):
        pltpu.async_copy(emb_hbm.at[idx_v.at[j]], rows_v, sem).wait()
        pltpu.sync_copy(rows_v, out_hbm.at[pl.ds(_mo(w * RPW + j * 128, 128), 128)])


def _sc_embed_gather(tokens_pad, embed):
    k = pl.kernel(
        _embed_gather_body,
        out_type=jax.ShapeDtypeStruct((NP, 16), jnp.float32),
        mesh=_MESH(),
        compiler_params=_SC_PARAMS,
        scratch_types=[
            pltpu.VMEM((RPW // 128, 128), jnp.int32),
            pltpu.VMEM((128, 16), jnp.float32),
            pltpu.SemaphoreType.DMA,
        ],
    )
    return k(tokens_pad, embed)


def _bin_degree_body(e3_hbm, hist_hbm, bsrc_hbm, bdst_hbm, cnt_hbm,
                     src_v, dst_v, stage_s, stage_d,
                     cursor_v, counts_v, ones_v, zb_v, sh_src, sh_dst,
                     hcur, sem):
    cid = lax.axis_index("c")
    sid = lax.axis_index("s")
    w = _wid()
    zeros16i = jnp.zeros((16,), jnp.int32)
    for i in range(8):
        ones_v[pl.ds(i * 16, 16)] = jnp.ones((16,), jnp.float32)
    for i in range(2):
        cursor_v[pl.ds(i * 16, 16)] = zeros16i
    for b in range(32):
        hcur[b] = 0

    def zero_body(i, _):
        zb_v[pl.ds(_mo(i * 16, 16), 16)] = jnp.zeros((16,), jnp.float32)
        return 0

    lax.fori_loop(0, (NP // 16) // 16, zero_body, 0)
    pltpu.sync_copy(zb_v, sh_src.at[pl.ds(_mo(sid * (NP // 16), 64), NP // 16)])
    pltpu.sync_copy(zb_v, sh_dst.at[pl.ds(_mo(sid * (NP // 16), 64), NP // 16)])
    plsc.subcore_barrier()

    def _cursor(b):
        return cursor_v[pl.ds(16 * (b // 16), 16)][b % 16]

    def flush_scan():
        for b in range(32):
            cur = _cursor(b)

            @pl.when(cur >= FLUSH)
            def _():
                hb = hcur[b]
                pltpu.sync_copy(
                    stage_s.at[pl.ds(b * STG, FLUSH)],
                    bsrc_hbm.at[pl.ds(_mo((w * 32 + b) * CAP + hb, 512), FLUSH)])
                pltpu.sync_copy(
                    stage_d.at[pl.ds(b * STG, FLUSH)],
                    bdst_hbm.at[pl.ds(_mo((w * 32 + b) * CAP + hb, 512), FLUSH)])
                hcur[b] = jnp.minimum(hb + FLUSH, CAP - FLUSH)
                for k in range(8):
                    v1 = stage_s[pl.ds(b * STG + FLUSH + k * 16, 16)]
                    stage_s[pl.ds(b * STG + k * 16, 16)] = v1
                    v2 = stage_d[pl.ds(b * STG + FLUSH + k * 16, 16)]
                    stage_d[pl.ds(b * STG + k * 16, 16)] = v2
                plsc.store_scatter(
                    cursor_v, [jnp.full((16,), b, jnp.int32)],
                    jnp.full((16,), cur - FLUSH, jnp.int32),
                    mask=_iota16() == 0)

    def chunk_body(j, _):
        rbase = w * EROWS_PW + j * ECH_ROWS
        pltpu.sync_copy(e3_hbm.at[0, pl.ds(_mo(rbase, 16), ECH_ROWS)], src_v)
        pltpu.sync_copy(e3_hbm.at[1, pl.ds(_mo(rbase, 16), ECH_ROWS)], dst_v)
        descs = []
        for r in range(ECH_ROWS):
            descs.append(pltpu.async_copy(
                ones_v, sh_src.at[src_v.at[r]], sem, add=True))
            descs.append(pltpu.async_copy(
                ones_v, sh_dst.at[dst_v.at[r]], sem, add=True))
        def vreg_body(v, _):
            ridx = jnp.full((16,), v // 8, jnp.int32)
            cidx = (v % 8) * 16 + _iota16()
            sv = plsc.load_gather(src_v, [ridx, cidx])
            dv = plsc.load_gather(dst_v, [ridx, cidx])
            binv = dv // BIN
            dloc = dv - binv * BIN
            occ, lastm = plsc.scan_count(binv)
            base = plsc.load_gather(cursor_v, [binv])
            fidx = binv * STG + base + occ - 1
            plsc.store_scatter(stage_s, [fidx], sv)
            plsc.store_scatter(stage_d, [fidx], dloc)
            plsc.store_scatter(cursor_v, [binv], base + occ, mask=lastm)

            @pl.when((v & 7) == 7)
            def _():
                c1 = jnp.max(cursor_v[pl.ds(0, 16)])
                c2 = jnp.max(cursor_v[pl.ds(16, 16)])

                @pl.when(jnp.maximum(c1, c2) >= FLUSH)
                def _():
                    flush_scan()
            return 0

        lax.fori_loop(0, ECH // 16, vreg_body, 0)
        for d in descs:
            d.wait()
        return 0

    lax.fori_loop(0, NCHUNK_E, chunk_body, 0)

    iota = _iota16()
    for b in range(32):
        cur0 = _cursor(b)
        pad = (-cur0) & 15
        pidx = b * STG + cur0 + iota
        plsc.store_scatter(stage_s, [pidx],
                           jnp.full((16,), NP - 1, jnp.int32),
                           mask=iota < pad)
        plsc.store_scatter(stage_d, [pidx],
                           jnp.full((16,), BIN - 1, jnp.int32),
                           mask=iota < pad)
        cur = cur0 + pad
        hb = hcur[b]
        pltpu.sync_copy(
            stage_s.at[pl.ds(b * STG, FLUSH)],
            bsrc_hbm.at[pl.ds(_mo((w * 32 + b) * CAP + hb, 512), FLUSH)])
        pltpu.sync_copy(
            stage_d.at[pl.ds(b * STG, FLUSH)],
            bdst_hbm.at[pl.ds(_mo((w * 32 + b) * CAP + hb, 512), FLUSH)])
        plsc.store_scatter(
            counts_v, [jnp.full((16,), b, jnp.int32)],
            jnp.full((16,), jnp.minimum(hb + cur, CAP), jnp.int32),
            mask=_iota16() == 0)
    pltpu.sync_copy(counts_v, cnt_hbm.at[pl.ds(_mo(w * 32, 32), 32)])

    plsc.subcore_barrier()

    @pl.when(sid == 0)
    def _():
        pltpu.sync_copy(sh_src, hist_hbm.at[cid, 0])
        pltpu.sync_copy(sh_dst, hist_hbm.at[cid, 1])


def _sc_bin_degrees(edges3):
    k = pl.kernel(
        _bin_degree_body,
        out_type=[
            jax.ShapeDtypeStruct((2, 2, NP), jnp.float32),
            jax.ShapeDtypeStruct((NW * 32 * CAP,), jnp.int32),
            jax.ShapeDtypeStruct((NW * 32 * CAP,), jnp.int32),
            jax.ShapeDtypeStruct((NW * 32,), jnp.int32),
        ],
        mesh=_MESH(),
        compiler_params=_SC_PARAMS,
        scratch_types=[
            pltpu.VMEM((ECH_ROWS, 128), jnp.int32),
            pltpu.VMEM((ECH_ROWS, 128), jnp.int32),
            pltpu.VMEM((32 * STG,), jnp.int32),
            pltpu.VMEM((32 * STG,), jnp.int32),
            pltpu.VMEM((32,), jnp.int32),
            pltpu.VMEM((32,), jnp.int32),
            pltpu.VMEM((128,), jnp.float32),
            pltpu.VMEM((NP // 16,), jnp.float32),
            pltpu.VMEM_SHARED((NP,), jnp.float32),
            pltpu.VMEM_SHARED((NP,), jnp.float32),
            pltpu.SMEM((32,), jnp.int32),
            pltpu.SemaphoreType.DMA,
        ],
    )
    return k(edges3)


def _seg_sum_body(bsrc_hbm, bdst_hbm, cnt_hbm, hn_hbm, agg_hbm,
                  esrc_v, edst_v, rows_v, acc_f, cnt_v, sem):
    t = _wid()
    iota = _iota16()
    pltpu.sync_copy(cnt_hbm, cnt_v)

    def zacc(i, _):
        acc_f[pl.ds(_mo(i * 16, 16), 16)] = jnp.zeros((16,), jnp.float32)
        return 0

    lax.fori_loop(0, BIN, zacc, 0)

    def group_body(g, _):
        dv = edst_v[pl.ds(_mo(g * 16, 16), 16)]
        for i in range(16):
            e16 = jnp.full((16,), 0, jnp.int32) + (g * 16 + i)
            d = dv[i]
            row = plsc.load_gather(rows_v, [e16, iota])
            a = acc_f[pl.ds(_mo(d * 16, 16), 16)]
            acc_f[pl.ds(_mo(d * 16, 16), 16)] = a + row
        return 0

    def prod_body(p, _):
        cnt = plsc.load_gather(
            cnt_v, [jnp.full((16,), p * 32 + t, jnp.int32)])[0]
        nch = (cnt + CCH - 1) // CCH

        def cbody(c, _):
            off = (p * 32 + t) * CAP + c * CCH
            pltpu.sync_copy(bsrc_hbm.at[pl.ds(_mo(off, 512), CCH)], esrc_v)
            pltpu.sync_copy(bdst_hbm.at[pl.ds(_mo(off, 512), CCH)], edst_v)

            def clampb(k, _):
                x = esrc_v[pl.ds(_mo(k * 16, 16), 16)]
                esrc_v[pl.ds(_mo(k * 16, 16), 16)] = jnp.clip(x, 0, NP - 1)
                return 0

            lax.fori_loop(0, CCH // 16, clampb, 0)
            for k in range(CCH // 128):
                pltpu.async_copy(
                    hn_hbm.at[esrc_v.at[pl.ds(k * 128, 128)]],
                    rows_v.at[pl.ds(k * 128, 128)], sem).wait()
            lax.fori_loop(0, jnp.minimum(CCH, cnt - c * CCH) // 16, group_body, 0)
            return 0

        lax.fori_loop(0, nch, cbody, 0)
        return 0

    lax.fori_loop(0, 32, prod_body, 0)

    pltpu.sync_copy(acc_f, agg_hbm.at[pl.ds(_mo(t * BIN * 16, 512), BIN * 16)])


def _sc_seg_sum(bsrc, bdst, cnts, hn_pad):
    k = pl.kernel(
        _seg_sum_body,
        out_type=jax.ShapeDtypeStruct((NP * 16,), jnp.float32),
        mesh=_MESH(),
        compiler_params=_SC_PARAMS,
        scratch_types=[
            pltpu.VMEM((CCH,), jnp.int32),
            pltpu.VMEM((CCH,), jnp.int32),
            pltpu.VMEM((CCH, 16), jnp.float32),
            pltpu.VMEM((BIN * 16,), jnp.float32),
            pltpu.VMEM((NW * 32,), jnp.int32),
            pltpu.SemaphoreType.DMA,
        ],
    )
    return k(bsrc, bdst, cnts, hn_pad)


def _seg_max_body(bsrc_hbm, bdst_hbm, cnt_hbm, hpa_hbm, hpb_hbm,
                  nga_hbm, ngb_hbm,
                  esrc_v, edst_v, rowsa_v, rowsb_v, acca_f, accb_f, cnt_v,
                  sem):
    t = _wid()
    iota = _iota16()
    pltpu.sync_copy(cnt_hbm, cnt_v)

    def zacc(i, _):
        acca_f[pl.ds(_mo(i * 16, 16), 16)] = jnp.zeros((16,), jnp.float32)
        accb_f[pl.ds(_mo(i * 16, 16), 16)] = jnp.zeros((16,), jnp.float32)
        return 0

    lax.fori_loop(0, BIN, zacc, 0)

    def group_body(g, _):
        dv = edst_v[pl.ds(_mo(g * 16, 16), 16)]
        for i in range(16):
            e16 = jnp.full((16,), 0, jnp.int32) + (g * 16 + i)
            d = dv[i]
            ra = plsc.load_gather(rowsa_v, [e16, iota])
            rb = plsc.load_gather(rowsb_v, [e16, iota])
            a = acca_f[pl.ds(_mo(d * 16, 16), 16)]
            acca_f[pl.ds(_mo(d * 16, 16), 16)] = jnp.maximum(a, ra)
            b = accb_f[pl.ds(_mo(d * 16, 16), 16)]
            accb_f[pl.ds(_mo(d * 16, 16), 16)] = jnp.maximum(b, rb)
        return 0

    def prod_body(p, _):
        cnt = plsc.load_gather(
            cnt_v, [jnp.full((16,), p * 32 + t, jnp.int32)])[0]
        nch = (cnt + CCH - 1) // CCH

        def cbody(c, _):
            off = (p * 32 + t) * CAP + c * CCH
            pltpu.sync_copy(bsrc_hbm.at[pl.ds(_mo(off, 512), CCH)], esrc_v)
            pltpu.sync_copy(bdst_hbm.at[pl.ds(_mo(off, 512), CCH)], edst_v)

            def clampb(k, _):
                x = esrc_v[pl.ds(_mo(k * 16, 16), 16)]
                esrc_v[pl.ds(_mo(k * 16, 16), 16)] = jnp.clip(x, 0, NP - 1)
                return 0

            lax.fori_loop(0, CCH // 16, clampb, 0)
            for k in range(CCH // 128):
                pltpu.async_copy(
                    hpa_hbm.at[esrc_v.at[pl.ds(k * 128, 128)]],
                    rowsa_v.at[pl.ds(k * 128, 128)], sem).wait()
                pltpu.async_copy(
                    hpb_hbm.at[esrc_v.at[pl.ds(k * 128, 128)]],
                    rowsb_v.at[pl.ds(k * 128, 128)], sem).wait()
            lax.fori_loop(0, jnp.minimum(CCH, cnt - c * CCH) // 16, group_body, 0)
            return 0

        lax.fori_loop(0, nch, cbody, 0)
        return 0

    lax.fori_loop(0, 32, prod_body, 0)

    pltpu.sync_copy(acca_f, nga_hbm.at[pl.ds(_mo(t * BIN * 16, 512), BIN * 16)])
    pltpu.sync_copy(accb_f, ngb_hbm.at[pl.ds(_mo(t * BIN * 16, 512), BIN * 16)])


def _sc_seg_max(bsrc, bdst, cnts, hpa_pad, hpb_pad):
    k = pl.kernel(
        _seg_max_body,
        out_type=[
            jax.ShapeDtypeStruct((NP * 16,), jnp.float32),
            jax.ShapeDtypeStruct((NP * 16,), jnp.float32),
        ],
        mesh=_MESH(),
        compiler_params=_SC_PARAMS,
        scratch_types=[
            pltpu.VMEM((CCH,), jnp.int32),
            pltpu.VMEM((CCH,), jnp.int32),
            pltpu.VMEM((CCH, 16), jnp.float32),
            pltpu.VMEM((CCH, 16), jnp.float32),
            pltpu.VMEM((BIN * 16,), jnp.float32),
            pltpu.VMEM((BIN * 16,), jnp.float32),
            pltpu.VMEM((NW * 32,), jnp.int32),
            pltpu.SemaphoreType.DMA,
        ],
    )
    return k(bsrc, bdst, cnts, hpa_pad, hpb_pad)


BLK = 2000


def _tail_body(h1_ref, neigh_ref, wself_ref, wneigh_ref, bneigh_ref,
               wlin_ref, blin_ref, out_ref):
    i = pl.program_id(0)
    h1 = h1_ref[...]
    neigh = neigh_ref[...]
    h2 = jnp.maximum(
        h1 @ wself_ref[...] + neigh @ wneigh_ref[...] + bneigh_ref[...], 0.0)
    h3 = jnp.maximum(h2 @ wlin_ref[...] + blin_ref[...], 0.0)
    part = jnp.sum(h3, axis=0, keepdims=True)

    @pl.when(i == 0)
    def _():
        out_ref[...] = jnp.zeros_like(out_ref)

    out_ref[0:1, :] += part


def _dense_tail(h1, neigh, Wself, Wneigh, bneigh, Wlin, blin):
    return pl.pallas_call(
        _tail_body,
        grid=(N // BLK,),
        in_specs=[
            pl.BlockSpec((BLK, 32), lambda i: (i, 0)),
            pl.BlockSpec((BLK, 32), lambda i: (i, 0)),
            pl.BlockSpec((32, 64), lambda i: (0, 0)),
            pl.BlockSpec((32, 64), lambda i: (0, 0)),
            pl.BlockSpec((64,), lambda i: (0,)),
            pl.BlockSpec((64, 64), lambda i: (0, 0)),
            pl.BlockSpec((64,), lambda i: (0,)),
        ],
        out_specs=pl.BlockSpec((8, 64), lambda i: (0, 0)),
        out_shape=jax.ShapeDtypeStruct((8, 64), jnp.float32),
    )(h1, neigh, Wself, Wneigh, bneigh, Wlin, blin)


def kernel(node_tokens, edge_index, embed, W1, b1, Wpool, bpool, Wself,
           Wneigh, bneigh, Wlin, blin):
    tokens_pad = jnp.pad(node_tokens.astype(jnp.int32), (0, NP - N))
    edges3 = jnp.pad(edge_index.astype(jnp.int32), ((0, 0), (0, EP - E)),
                     constant_values=N).reshape(2, EP // 128, 128)
    h_pad = _sc_embed_gather(tokens_pad, embed)
    hist, bsrc, bdst, cnts = _sc_bin_degrees(edges3)
    deg = hist[0] + hist[1]
    out_deg = deg[0, :N]
    in_deg = deg[1, :N]

    c_src_pad = jax.lax.rsqrt(jnp.clip(deg[0], 1.0))
    c_dst = jax.lax.rsqrt(jnp.clip(in_deg, 1.0))
    hn_pad = h_pad * c_src_pad[:, None]
    hn_pad = hn_pad.at[NP - 1].set(0.0)

    agg = _sc_seg_sum(bsrc, bdst, cnts, hn_pad).reshape(NP, 16)[:N]
    h1 = jax.nn.relu((agg * c_dst[:, None]) @ W1 + b1)
    hp = jax.nn.relu(h1 @ Wpool + bpool)

    hp_pad = jnp.pad(hp, ((0, NP - N), (0, 0)))
    nga, ngb = _sc_seg_max(bsrc, bdst, cnts, hp_pad[:, :16].copy(),
                           hp_pad[:, 16:].copy())
    neigh = jnp.concatenate(
        [nga.reshape(NP, 16)[:N], ngb.reshape(NP, 16)[:N]], axis=1)

    part = _dense_tail(h1, neigh, Wself, Wneigh, bneigh, Wlin, blin)
    return jnp.sum(part, axis=0, keepdims=True)


# speedup vs baseline: 9.0772x; 1.0062x over previous
import functools

import jax
import jax.numpy as jnp
from jax import lax
from jax.experimental import pallas as pl
from jax.experimental.pallas import tpu as pltpu
from jax.experimental.pallas import tpu_sc as plsc

N = 100000
E = 3200000
VOCAB = 100000
NP = 102400
EP = 3276800
NW = 32
RPW = NP // NW
BIN = RPW
ECH = 2048
ECH_ROWS = ECH // 128
EROWS_PW = EP // NW // 128
NCHUNK_E = EP // NW // ECH
CAP = 8192
STG = 1024
FLUSH = 512
CCH = 512

_MESH = functools.partial(plsc.VectorSubcoreMesh,
                          core_axis_name="c", subcore_axis_name="s")
_SC_PARAMS = pltpu.CompilerParams(use_tc_tiling_on_sc=False,
                                  needs_layout_passes=False)


def _wid():
    return lax.axis_index("s") * 2 + lax.axis_index("c")


def _iota16():
    return lax.iota(jnp.int32, 16)


def _mo(x, n):
    return pl.multiple_of(x, n)


def _embed_gather_body(tok_hbm, emb_hbm, out_hbm, idx_v, rows_v, sem):
    w = _wid()
    for j in range(RPW // 128):
        pltpu.sync_copy(tok_hbm.at[pl.ds(_mo(w * RPW + j * 128, 128), 128)], idx_v.at[j])
    for j in range(RPW // 128):
        pltpu.async_copy(emb_hbm.at[idx_v.at[j]], rows_v, sem).wait()
        pltpu.sync_copy(rows_v, out_hbm.at[pl.ds(_mo(w * RPW + j * 128, 128), 128)])


def _sc_embed_gather(tokens_pad, embed):
    k = pl.kernel(
        _embed_gather_body,
        out_type=jax.ShapeDtypeStruct((NP, 16), jnp.float32),
        mesh=_MESH(),
        compiler_params=_SC_PARAMS,
        scratch_types=[
            pltpu.VMEM((RPW // 128, 128), jnp.int32),
            pltpu.VMEM((128, 16), jnp.float32),
            pltpu.SemaphoreType.DMA,
        ],
    )
    return k(tokens_pad, embed)


def _bin_degree_body(e3_hbm, hist_hbm, bsrc_hbm, bdst_hbm, cnt_hbm,
                     src_v, dst_v, stage_s, stage_d,
                     cursor_v, counts_v, ones_v, zb_v, sh_src, sh_dst,
                     hcur, sem):
    cid = lax.axis_index("c")
    sid = lax.axis_index("s")
    w = _wid()
    zeros16i = jnp.zeros((16,), jnp.int32)
    for i in range(8):
        ones_v[pl.ds(i * 16, 16)] = jnp.ones((16,), jnp.float32)
    for i in range(2):
        cursor_v[pl.ds(i * 16, 16)] = zeros16i
    for b in range(32):
        hcur[b] = 0

    def zero_body(i, _):
        zb_v[pl.ds(_mo(i * 16, 16), 16)] = jnp.zeros((16,), jnp.float32)
        return 0

    lax.fori_loop(0, (NP // 16) // 16, zero_body, 0)
    pltpu.sync_copy(zb_v, sh_src.at[pl.ds(_mo(sid * (NP // 16), 64), NP // 16)])
    pltpu.sync_copy(zb_v, sh_dst.at[pl.ds(_mo(sid * (NP // 16), 64), NP // 16)])
    plsc.subcore_barrier()

    def _cursor(b):
        return cursor_v[pl.ds(16 * (b // 16), 16)][b % 16]

    def flush_scan():
        for b in range(32):
            cur = _cursor(b)

            @pl.when(cur >= FLUSH)
            def _():
                hb = hcur[b]
                pltpu.sync_copy(
                    stage_s.at[pl.ds(b * STG, FLUSH)],
                    bsrc_hbm.at[pl.ds(_mo((w * 32 + b) * CAP + hb, 512), FLUSH)])
                pltpu.sync_copy(
                    stage_d.at[pl.ds(b * STG, FLUSH)],
                    bdst_hbm.at[pl.ds(_mo((w * 32 + b) * CAP + hb, 512), FLUSH)])
                hcur[b] = jnp.minimum(hb + FLUSH, CAP - FLUSH)
                for k in range(8):
                    v1 = stage_s[pl.ds(b * STG + FLUSH + k * 16, 16)]
                    stage_s[pl.ds(b * STG + k * 16, 16)] = v1
                    v2 = stage_d[pl.ds(b * STG + FLUSH + k * 16, 16)]
                    stage_d[pl.ds(b * STG + k * 16, 16)] = v2
                plsc.store_scatter(
                    cursor_v, [jnp.full((16,), b, jnp.int32)],
                    jnp.full((16,), cur - FLUSH, jnp.int32),
                    mask=_iota16() == 0)

    def chunk_body(j, _):
        rbase = w * EROWS_PW + j * ECH_ROWS
        pltpu.sync_copy(e3_hbm.at[0, pl.ds(_mo(rbase, 16), ECH_ROWS)], src_v)
        pltpu.sync_copy(e3_hbm.at[1, pl.ds(_mo(rbase, 16), ECH_ROWS)], dst_v)
        descs = []
        for r in range(ECH_ROWS):
            descs.append(pltpu.async_copy(
                ones_v, sh_src.at[src_v.at[r]], sem, add=True))
            descs.append(pltpu.async_copy(
                ones_v, sh_dst.at[dst_v.at[r]], sem, add=True))
        def vreg_body(v, _):
            ridx = jnp.full((16,), v // 8, jnp.int32)
            cidx = (v % 8) * 16 + _iota16()
            sv = plsc.load_gather(src_v, [ridx, cidx])
            dv = plsc.load_gather(dst_v, [ridx, cidx])
            binv = dv // BIN
            dloc = dv - binv * BIN
            occ, lastm = plsc.scan_count(binv)
            base = plsc.load_gather(cursor_v, [binv])
            fidx = binv * STG + base + occ - 1
            plsc.store_scatter(stage_s, [fidx], sv)
            plsc.store_scatter(stage_d, [fidx], dloc)
            plsc.store_scatter(cursor_v, [binv], base + occ, mask=lastm)

            @pl.when((v & 7) == 7)
            def _():
                c1 = jnp.max(cursor_v[pl.ds(0, 16)])
                c2 = jnp.max(cursor_v[pl.ds(16, 16)])

                @pl.when(jnp.maximum(c1, c2) >= FLUSH)
                def _():
                    flush_scan()
            return 0

        lax.fori_loop(0, ECH // 16, vreg_body, 0)
        for d in descs:
            d.wait()
        return 0

    lax.fori_loop(0, NCHUNK_E, chunk_body, 0)

    iota = _iota16()
    for b in range(32):
        cur0 = _cursor(b)
        pad = (-cur0) & 15
        pidx = b * STG + cur0 + iota
        plsc.store_scatter(stage_s, [pidx],
                           jnp.full((16,), NP - 1, jnp.int32),
                           mask=iota < pad)
        plsc.store_scatter(stage_d, [pidx],
                           jnp.full((16,), BIN - 1, jnp.int32),
                           mask=iota < pad)
        cur = cur0 + pad
        hb = hcur[b]
        pltpu.sync_copy(
            stage_s.at[pl.ds(b * STG, FLUSH)],
            bsrc_hbm.at[pl.ds(_mo((w * 32 + b) * CAP + hb, 512), FLUSH)])
        pltpu.sync_copy(
            stage_d.at[pl.ds(b * STG, FLUSH)],
            bdst_hbm.at[pl.ds(_mo((w * 32 + b) * CAP + hb, 512), FLUSH)])
        plsc.store_scatter(
            counts_v, [jnp.full((16,), b, jnp.int32)],
            jnp.full((16,), jnp.minimum(hb + cur, CAP), jnp.int32),
            mask=_iota16() == 0)
    pltpu.sync_copy(counts_v, cnt_hbm.at[pl.ds(_mo(w * 32, 32), 32)])

    plsc.subcore_barrier()

    @pl.when(sid == 0)
    def _():
        pltpu.sync_copy(sh_src, hist_hbm.at[cid, 0])
        pltpu.sync_copy(sh_dst, hist_hbm.at[cid, 1])


def _sc_bin_degrees(edges3):
    k = pl.kernel(
        _bin_degree_body,
        out_type=[
            jax.ShapeDtypeStruct((2, 2, NP), jnp.float32),
            jax.ShapeDtypeStruct((NW * 32 * CAP,), jnp.int32),
            jax.ShapeDtypeStruct((NW * 32 * CAP,), jnp.int32),
            jax.ShapeDtypeStruct((NW * 32,), jnp.int32),
        ],
        mesh=_MESH(),
        compiler_params=_SC_PARAMS,
        scratch_types=[
            pltpu.VMEM((ECH_ROWS, 128), jnp.int32),
            pltpu.VMEM((ECH_ROWS, 128), jnp.int32),
            pltpu.VMEM((32 * STG,), jnp.int32),
            pltpu.VMEM((32 * STG,), jnp.int32),
            pltpu.VMEM((32,), jnp.int32),
            pltpu.VMEM((32,), jnp.int32),
            pltpu.VMEM((128,), jnp.float32),
            pltpu.VMEM((NP // 16,), jnp.float32),
            pltpu.VMEM_SHARED((NP,), jnp.float32),
            pltpu.VMEM_SHARED((NP,), jnp.float32),
            pltpu.SMEM((32,), jnp.int32),
            pltpu.SemaphoreType.DMA,
        ],
    )
    return k(edges3)


def _seg_sum_body(bsrc_hbm, bdst_hbm, cnt_hbm, hn_hbm, agg_hbm,
                  esrc_v, edst_v, rows_v, acc_f, cnt_v, sem):
    t = _wid()
    iota = _iota16()
    pltpu.sync_copy(cnt_hbm, cnt_v)

    def zacc(i, _):
        acc_f[pl.ds(_mo(i * 16, 16), 16)] = jnp.zeros((16,), jnp.float32)
        return 0

    lax.fori_loop(0, BIN, zacc, 0)

    def group_body(g, _):
        dv = edst_v[pl.ds(_mo(g * 16, 16), 16)]
        for i in range(16):
            e16 = jnp.full((16,), 0, jnp.int32) + (g * 16 + i)
            d = dv[i]
            row = plsc.load_gather(rows_v, [e16, iota])
            a = acc_f[pl.ds(_mo(d * 16, 16), 16)]
            acc_f[pl.ds(_mo(d * 16, 16), 16)] = a + row
        return 0

    def prod_body(p, _):
        cnt = plsc.load_gather(
            cnt_v, [jnp.full((16,), p * 32 + t, jnp.int32)])[0]
        nch = (cnt + CCH - 1) // CCH

        def cbody(c, _):
            off = (p * 32 + t) * CAP + c * CCH
            pltpu.sync_copy(bsrc_hbm.at[pl.ds(_mo(off, 512), CCH)], esrc_v)
            pltpu.sync_copy(bdst_hbm.at[pl.ds(_mo(off, 512), CCH)], edst_v)

            def clampb(k, _):
                x = esrc_v[pl.ds(_mo(k * 16, 16), 16)]
                esrc_v[pl.ds(_mo(k * 16, 16), 16)] = jnp.clip(x, 0, NP - 1)
                return 0

            lax.fori_loop(0, CCH // 16, clampb, 0)
            for k in range(CCH // 128):
                pltpu.async_copy(
                    hn_hbm.at[esrc_v.at[pl.ds(k * 128, 128)]],
                    rows_v.at[pl.ds(k * 128, 128)], sem).wait()
            lax.fori_loop(0, jnp.minimum(CCH, cnt - c * CCH) // 16, group_body, 0)
            return 0

        lax.fori_loop(0, nch, cbody, 0)
        return 0

    lax.fori_loop(0, 32, prod_body, 0)

    pltpu.sync_copy(acc_f, agg_hbm.at[pl.ds(_mo(t * BIN * 16, 512), BIN * 16)])


def _sc_seg_sum(bsrc, bdst, cnts, hn_pad):
    k = pl.kernel(
        _seg_sum_body,
        out_type=jax.ShapeDtypeStruct((NP * 16,), jnp.float32),
        mesh=_MESH(),
        compiler_params=_SC_PARAMS,
        scratch_types=[
            pltpu.VMEM((CCH,), jnp.int32),
            pltpu.VMEM((CCH,), jnp.int32),
            pltpu.VMEM((CCH, 16), jnp.float32),
            pltpu.VMEM((BIN * 16,), jnp.float32),
            pltpu.VMEM((NW * 32,), jnp.int32),
            pltpu.SemaphoreType.DMA,
        ],
    )
    return k(bsrc, bdst, cnts, hn_pad)


def _seg_max_body(bsrc_hbm, bdst_hbm, cnt_hbm, hpa_hbm, hpb_hbm,
                  nga_hbm, ngb_hbm,
                  esrc_v, edst_v, rowsa_v, rowsb_v, acca_f, accb_f, cnt_v,
                  sem):
    t = _wid()
    iota = _iota16()
    pltpu.sync_copy(cnt_hbm, cnt_v)

    def zacc(i, _):
        acca_f[pl.ds(_mo(i * 16, 16), 16)] = jnp.zeros((16,), jnp.float32)
        accb_f[pl.ds(_mo(i * 16, 16), 16)] = jnp.zeros((16,), jnp.float32)
        return 0

    lax.fori_loop(0, BIN, zacc, 0)

    def group_body(g, _):
        dv = edst_v[pl.ds(_mo(g * 16, 16), 16)]
        for i in range(16):
            e16 = jnp.full((16,), 0, jnp.int32) + (g * 16 + i)
            d = dv[i]
            ra = plsc.load_gather(rowsa_v, [e16, iota])
            rb = plsc.load_gather(rowsb_v, [e16, iota])
            a = acca_f[pl.ds(_mo(d * 16, 16), 16)]
            acca_f[pl.ds(_mo(d * 16, 16), 16)] = jnp.maximum(a, ra)
            b = accb_f[pl.ds(_mo(d * 16, 16), 16)]
            accb_f[pl.ds(_mo(d * 16, 16), 16)] = jnp.maximum(b, rb)
        return 0

    def prod_body(p, _):
        cnt = plsc.load_gather(
            cnt_v, [jnp.full((16,), p * 32 + t, jnp.int32)])[0]
        nch = (cnt + CCH - 1) // CCH

        def cbody(c, _):
            off = (p * 32 + t) * CAP + c * CCH
            pltpu.sync_copy(bsrc_hbm.at[pl.ds(_mo(off, 512), CCH)], esrc_v)
            pltpu.sync_copy(bdst_hbm.at[pl.ds(_mo(off, 512), CCH)], edst_v)

            def clampb(k, _):
                x = esrc_v[pl.ds(_mo(k * 16, 16), 16)]
                esrc_v[pl.ds(_mo(k * 16, 16), 16)] = jnp.clip(x, 0, NP - 1)
                return 0

            lax.fori_loop(0, CCH // 16, clampb, 0)
            for k in range(CCH // 128):
                pltpu.async_copy(
                    hpa_hbm.at[esrc_v.at[pl.ds(k * 128, 128)]],
                    rowsa_v.at[pl.ds(k * 128, 128)], sem).wait()
                pltpu.async_copy(
                    hpb_hbm.at[esrc_v.at[pl.ds(k * 128, 128)]],
                    rowsb_v.at[pl.ds(k * 128, 128)], sem).wait()
            lax.fori_loop(0, jnp.minimum(CCH, cnt - c * CCH) // 16, group_body, 0)
            return 0

        lax.fori_loop(0, nch, cbody, 0)
        return 0

    lax.fori_loop(0, 32, prod_body, 0)

    pltpu.sync_copy(acca_f, nga_hbm.at[pl.ds(_mo(t * BIN * 16, 512), BIN * 16)])
    pltpu.sync_copy(accb_f, ngb_hbm.at[pl.ds(_mo(t * BIN * 16, 512), BIN * 16)])


def _sc_seg_max(bsrc, bdst, cnts, hpa_pad, hpb_pad):
    k = pl.kernel(
        _seg_max_body,
        out_type=[
            jax.ShapeDtypeStruct((NP * 16,), jnp.float32),
            jax.ShapeDtypeStruct((NP * 16,), jnp.float32),
        ],
        mesh=_MESH(),
        compiler_params=_SC_PARAMS,
        scratch_types=[
            pltpu.VMEM((CCH,), jnp.int32),
            pltpu.VMEM((CCH,), jnp.int32),
            pltpu.VMEM((CCH, 16), jnp.float32),
            pltpu.VMEM((CCH, 16), jnp.float32),
            pltpu.VMEM((BIN * 16,), jnp.float32),
            pltpu.VMEM((BIN * 16,), jnp.float32),
            pltpu.VMEM((NW * 32,), jnp.int32),
            pltpu.SemaphoreType.DMA,
        ],
    )
    return k(bsrc, bdst, cnts, hpa_pad, hpb_pad)


TBLK = 2048


def _prep_body(hist_ref, h_ref, hn_ref):
    i = pl.program_id(0)
    deg_s = hist_ref[0, :] + hist_ref[2, :]
    c_src = jax.lax.rsqrt(jnp.maximum(deg_s, 1.0))
    row = i * TBLK + jax.lax.broadcasted_iota(jnp.int32, (TBLK, 1), 0)
    hn = h_ref[...] * c_src[:, None]
    hn_ref[...] = jnp.where(row < N, hn, 0.0)


def _tc_prep(hist4, h_pad):
    return pl.pallas_call(
        _prep_body,
        grid=(NP // TBLK,),
        in_specs=[
            pl.BlockSpec((4, TBLK), lambda i: (0, i)),
            pl.BlockSpec((TBLK, 16), lambda i: (i, 0)),
        ],
        out_specs=pl.BlockSpec((TBLK, 16), lambda i: (i, 0)),
        out_shape=jax.ShapeDtypeStruct((NP, 16), jnp.float32),
    )(hist4, h_pad)


def _mid_body(hist_ref, agg_ref, w1_ref, b1_ref, wp_ref, bp_ref,
              h1_ref, hpa_ref, hpb_ref):
    i = pl.program_id(0)
    deg_d = hist_ref[1, :] + hist_ref[3, :]
    c_dst = jax.lax.rsqrt(jnp.maximum(deg_d, 1.0))
    aggc = agg_ref[...] * c_dst[:, None]
    h1 = jnp.maximum(aggc @ w1_ref[...] + b1_ref[...], 0.0)
    hp = jnp.maximum(h1 @ wp_ref[...] + bp_ref[...], 0.0)
    row = i * TBLK + jax.lax.broadcasted_iota(jnp.int32, (TBLK, 1), 0)
    hp = jnp.where(row < N, hp, 0.0)
    h1_ref[...] = h1
    hpa_ref[...] = hp[:, :16]
    hpb_ref[...] = hp[:, 16:]


def _tc_mid(hist4, agg_pad, W1, b1, Wpool, bpool):
    return pl.pallas_call(
        _mid_body,
        grid=(NP // TBLK,),
        in_specs=[
            pl.BlockSpec((4, TBLK), lambda i: (0, i)),
            pl.BlockSpec((TBLK, 16), lambda i: (i, 0)),
            pl.BlockSpec((16, 32), lambda i: (0, 0)),
            pl.BlockSpec((32,), lambda i: (0,)),
            pl.BlockSpec((32, 32), lambda i: (0, 0)),
            pl.BlockSpec((32,), lambda i: (0,)),
        ],
        out_specs=[
            pl.BlockSpec((TBLK, 32), lambda i: (i, 0)),
            pl.BlockSpec((TBLK, 16), lambda i: (i, 0)),
            pl.BlockSpec((TBLK, 16), lambda i: (i, 0)),
        ],
        out_shape=[
            jax.ShapeDtypeStruct((NP, 32), jnp.float32),
            jax.ShapeDtypeStruct((NP, 16), jnp.float32),
            jax.ShapeDtypeStruct((NP, 16), jnp.float32),
        ],
    )(hist4, agg_pad, W1, b1, Wpool, bpool)


BLK = 2000


def _tail_body(h1_ref, neigh_ref, wself_ref, wneigh_ref, bneigh_ref,
               wlin_ref, blin_ref, out_ref):
    i = pl.program_id(0)
    h1 = h1_ref[...]
    neigh = neigh_ref[...]
    h2 = jnp.maximum(
        h1 @ wself_ref[...] + neigh @ wneigh_ref[...] + bneigh_ref[...], 0.0)
    h3 = jnp.maximum(h2 @ wlin_ref[...] + blin_ref[...], 0.0)
    part = jnp.sum(h3, axis=0, keepdims=True)

    @pl.when(i == 0)
    def _():
        out_ref[...] = jnp.zeros_like(out_ref)

    out_ref[0:1, :] += part


def _dense_tail(h1, neigh, Wself, Wneigh, bneigh, Wlin, blin):
    return pl.pallas_call(
        _tail_body,
        grid=(N // BLK,),
        in_specs=[
            pl.BlockSpec((BLK, 32), lambda i: (i, 0)),
            pl.BlockSpec((BLK, 32), lambda i: (i, 0)),
            pl.BlockSpec((32, 64), lambda i: (0, 0)),
            pl.BlockSpec((32, 64), lambda i: (0, 0)),
            pl.BlockSpec((64,), lambda i: (0,)),
            pl.BlockSpec((64, 64), lambda i: (0, 0)),
            pl.BlockSpec((64,), lambda i: (0,)),
        ],
        out_specs=pl.BlockSpec((8, 64), lambda i: (0, 0)),
        out_shape=jax.ShapeDtypeStruct((8, 64), jnp.float32),
    )(h1, neigh, Wself, Wneigh, bneigh, Wlin, blin)


def kernel(node_tokens, edge_index, embed, W1, b1, Wpool, bpool, Wself,
           Wneigh, bneigh, Wlin, blin):
    tokens_pad = jnp.pad(node_tokens.astype(jnp.int32), (0, NP - N))
    edges3 = jnp.pad(edge_index.astype(jnp.int32), ((0, 0), (0, EP - E)),
                     constant_values=N).reshape(2, EP // 128, 128)
    h_pad = _sc_embed_gather(tokens_pad, embed)
    hist, bsrc, bdst, cnts = _sc_bin_degrees(edges3)
    hist4 = hist.reshape(4, NP)
    hn_pad = _tc_prep(hist4, h_pad)

    agg_pad = _sc_seg_sum(bsrc, bdst, cnts, hn_pad).reshape(NP, 16)
    h1_pad, hpa_pad, hpb_pad = _tc_mid(hist4, agg_pad, W1, b1, Wpool, bpool)

    nga, ngb = _sc_seg_max(bsrc, bdst, cnts, hpa_pad, hpb_pad)
    neigh = jnp.concatenate(
        [nga.reshape(NP, 16)[:N], ngb.reshape(NP, 16)[:N]], axis=1)

    part = _dense_tail(h1_pad[:N], neigh, Wself, Wneigh, bneigh, Wlin, blin)
    return jnp.sum(part, axis=0, keepdims=True)


# speedup vs baseline: 12.0752x; 1.3303x over previous
import functools

import jax
import jax.numpy as jnp
from jax import lax
from jax.experimental import pallas as pl
from jax.experimental.pallas import tpu as pltpu
from jax.experimental.pallas import tpu_sc as plsc

N = 100000
E = 3200000
VOCAB = 100000
NP = 102400
EP = 3276800
NW = 32
RPW = NP // NW
BIN = RPW
ECH = 2048
ECH_ROWS = ECH // 128
EROWS_PW = EP // NW // 128
NCHUNK_E = EP // NW // ECH
CAP = 8192
STG = 1024
FLUSH = 512
CCH = 512

_MESH = functools.partial(plsc.VectorSubcoreMesh,
                          core_axis_name="c", subcore_axis_name="s")
_SC_PARAMS = pltpu.CompilerParams(use_tc_tiling_on_sc=False,
                                  needs_layout_passes=False)


def _wid():
    return lax.axis_index("s") * 2 + lax.axis_index("c")


def _iota16():
    return lax.iota(jnp.int32, 16)


def _mo(x, n):
    return pl.multiple_of(x, n)


def _embed_gather_body(tok_hbm, emb_hbm, out_hbm, idx_v, rows_v, sem):
    w = _wid()
    for j in range(RPW // 128):
        pltpu.sync_copy(tok_hbm.at[pl.ds(_mo(w * RPW + j * 128, 128), 128)], idx_v.at[j])
    for j in range(RPW // 128):
        pltpu.async_copy(emb_hbm.at[idx_v.at[j]], rows_v, sem).wait()
        pltpu.sync_copy(rows_v, out_hbm.at[pl.ds(_mo(w * RPW + j * 128, 128), 128)])


def _sc_embed_gather(tokens_pad, embed):
    k = pl.kernel(
        _embed_gather_body,
        out_type=jax.ShapeDtypeStruct((NP, 16), jnp.float32),
        mesh=_MESH(),
        compiler_params=_SC_PARAMS,
        scratch_types=[
            pltpu.VMEM((RPW // 128, 128), jnp.int32),
            pltpu.VMEM((128, 16), jnp.float32),
            pltpu.SemaphoreType.DMA,
        ],
    )
    return k(tokens_pad, embed)


def _bin_degree_body(e3_hbm, hist_hbm, bsrc_hbm, bdst_hbm, cnt_hbm,
                     src_v, dst_v, stage_s, stage_d,
                     cursor_v, counts_v, ones_v, zb_v, sh_src, sh_dst,
                     hcur, sem):
    cid = lax.axis_index("c")
    sid = lax.axis_index("s")
    w = _wid()
    zeros16i = jnp.zeros((16,), jnp.int32)
    for i in range(8):
        ones_v[pl.ds(i * 16, 16)] = jnp.ones((16,), jnp.float32)
    for i in range(2):
        cursor_v[pl.ds(i * 16, 16)] = zeros16i
    for b in range(32):
        hcur[b] = 0

    def zero_body(i, _):
        zb_v[pl.ds(_mo(i * 16, 16), 16)] = jnp.zeros((16,), jnp.float32)
        return 0

    lax.fori_loop(0, (NP // 16) // 16, zero_body, 0)
    pltpu.sync_copy(zb_v, sh_src.at[pl.ds(_mo(sid * (NP // 16), 64), NP // 16)])
    pltpu.sync_copy(zb_v, sh_dst.at[pl.ds(_mo(sid * (NP // 16), 64), NP // 16)])
    plsc.subcore_barrier()

    def _cursor(b):
        return cursor_v[pl.ds(16 * (b // 16), 16)][b % 16]

    def flush_scan():
        for b in range(32):
            cur = _cursor(b)

            @pl.when(cur >= FLUSH)
            def _():
                hb = hcur[b]
                pltpu.sync_copy(
                    stage_s.at[pl.ds(b * STG, FLUSH)],
                    bsrc_hbm.at[pl.ds(_mo((w * 32 + b) * CAP + hb, 512), FLUSH)])
                pltpu.sync_copy(
                    stage_d.at[pl.ds(b * STG, FLUSH)],
                    bdst_hbm.at[pl.ds(_mo((w * 32 + b) * CAP + hb, 512), FLUSH)])
                hcur[b] = jnp.minimum(hb + FLUSH, CAP - FLUSH)
                for k in range(8):
                    v1 = stage_s[pl.ds(b * STG + FLUSH + k * 16, 16)]
                    stage_s[pl.ds(b * STG + k * 16, 16)] = v1
                    v2 = stage_d[pl.ds(b * STG + FLUSH + k * 16, 16)]
                    stage_d[pl.ds(b * STG + k * 16, 16)] = v2
                plsc.store_scatter(
                    cursor_v, [jnp.full((16,), b, jnp.int32)],
                    jnp.full((16,), cur - FLUSH, jnp.int32),
                    mask=_iota16() == 0)

    def chunk_body(j, _):
        rbase = w * EROWS_PW + j * ECH_ROWS
        pltpu.sync_copy(e3_hbm.at[0, pl.ds(_mo(rbase, 16), ECH_ROWS)], src_v)
        pltpu.sync_copy(e3_hbm.at[1, pl.ds(_mo(rbase, 16), ECH_ROWS)], dst_v)
        descs = []
        for r in range(ECH_ROWS):
            descs.append(pltpu.async_copy(
                ones_v, sh_src.at[src_v.at[r]], sem, add=True))
            descs.append(pltpu.async_copy(
                ones_v, sh_dst.at[dst_v.at[r]], sem, add=True))
        def vreg_body(v, _):
            ridx = jnp.full((16,), v // 8, jnp.int32)
            cidx = (v % 8) * 16 + _iota16()
            sv = plsc.load_gather(src_v, [ridx, cidx])
            dv = plsc.load_gather(dst_v, [ridx, cidx])
            binv = dv // BIN
            dloc = dv - binv * BIN
            occ, lastm = plsc.scan_count(binv)
            base = plsc.load_gather(cursor_v, [binv])
            fidx = binv * STG + base + occ - 1
            plsc.store_scatter(stage_s, [fidx], sv)
            plsc.store_scatter(stage_d, [fidx], dloc)
            plsc.store_scatter(cursor_v, [binv], base + occ, mask=lastm)

            @pl.when((v & 7) == 7)
            def _():
                c1 = jnp.max(cursor_v[pl.ds(0, 16)])
                c2 = jnp.max(cursor_v[pl.ds(16, 16)])

                @pl.when(jnp.maximum(c1, c2) >= FLUSH)
                def _():
                    flush_scan()
            return 0

        lax.fori_loop(0, ECH // 16, vreg_body, 0)
        for d in descs:
            d.wait()
        return 0

    lax.fori_loop(0, NCHUNK_E, chunk_body, 0)

    iota = _iota16()
    for b in range(32):
        cur0 = _cursor(b)
        pad = (-cur0) & 15
        pidx = b * STG + cur0 + iota
        plsc.store_scatter(stage_s, [pidx],
                           jnp.full((16,), NP - 1, jnp.int32),
                           mask=iota < pad)
        plsc.store_scatter(stage_d, [pidx],
                           jnp.full((16,), BIN - 1, jnp.int32),
                           mask=iota < pad)
        cur = cur0 + pad
        hb = hcur[b]
        pltpu.sync_copy(
            stage_s.at[pl.ds(b * STG, FLUSH)],
            bsrc_hbm.at[pl.ds(_mo((w * 32 + b) * CAP + hb, 512), FLUSH)])
        pltpu.sync_copy(
            stage_d.at[pl.ds(b * STG, FLUSH)],
            bdst_hbm.at[pl.ds(_mo((w * 32 + b) * CAP + hb, 512), FLUSH)])
        plsc.store_scatter(
            counts_v, [jnp.full((16,), b, jnp.int32)],
            jnp.full((16,), jnp.minimum(hb + cur, CAP), jnp.int32),
            mask=_iota16() == 0)
    pltpu.sync_copy(counts_v, cnt_hbm.at[pl.ds(_mo(w * 32, 32), 32)])

    plsc.subcore_barrier()

    @pl.when(sid == 0)
    def _():
        pltpu.sync_copy(sh_src, hist_hbm.at[cid, 0])
        pltpu.sync_copy(sh_dst, hist_hbm.at[cid, 1])


def _sc_bin_degrees(edges3):
    k = pl.kernel(
        _bin_degree_body,
        out_type=[
            jax.ShapeDtypeStruct((2, 2, NP), jnp.float32),
            jax.ShapeDtypeStruct((NW * 32 * CAP,), jnp.int32),
            jax.ShapeDtypeStruct((NW * 32 * CAP,), jnp.int32),
            jax.ShapeDtypeStruct((NW * 32,), jnp.int32),
        ],
        mesh=_MESH(),
        compiler_params=_SC_PARAMS,
        scratch_types=[
            pltpu.VMEM((ECH_ROWS, 128), jnp.int32),
            pltpu.VMEM((ECH_ROWS, 128), jnp.int32),
            pltpu.VMEM((32 * STG,), jnp.int32),
            pltpu.VMEM((32 * STG,), jnp.int32),
            pltpu.VMEM((32,), jnp.int32),
            pltpu.VMEM((32,), jnp.int32),
            pltpu.VMEM((128,), jnp.float32),
            pltpu.VMEM((NP // 16,), jnp.float32),
            pltpu.VMEM_SHARED((NP,), jnp.float32),
            pltpu.VMEM_SHARED((NP,), jnp.float32),
            pltpu.SMEM((32,), jnp.int32),
            pltpu.SemaphoreType.DMA,
        ],
    )
    return k(edges3)


def _seg_sum_body(bsrc_hbm, bdst_hbm, cnt_hbm, hn_hbm, agg_hbm,
                  esrc_v, edst_v, rows_v, acc_f, cnt_v, sem):
    t = _wid()
    iota = _iota16()
    pltpu.sync_copy(cnt_hbm, cnt_v)

    def zacc(i, _):
        acc_f[pl.ds(_mo(i * 16, 16), 16)] = jnp.zeros((16,), jnp.float32)
        return 0

    lax.fori_loop(0, BIN, zacc, 0)

    def group_body(g, _):
        dv = edst_v[pl.ds(_mo(g * 16, 16), 16)]
        for i in range(16):
            e16 = jnp.full((16,), 0, jnp.int32) + (g * 16 + i)
            d = dv[i]
            row = plsc.load_gather(rows_v, [e16, iota])
            a = acc_f[pl.ds(_mo(d * 16, 16), 16)]
            acc_f[pl.ds(_mo(d * 16, 16), 16)] = a + row
        return 0

    def prod_body(p, _):
        cnt = plsc.load_gather(
            cnt_v, [jnp.full((16,), p * 32 + t, jnp.int32)])[0]
        nch = (cnt + CCH - 1) // CCH

        def cbody(c, _):
            off = (p * 32 + t) * CAP + c * CCH
            pltpu.sync_copy(bsrc_hbm.at[pl.ds(_mo(off, 512), CCH)], esrc_v)
            pltpu.sync_copy(bdst_hbm.at[pl.ds(_mo(off, 512), CCH)], edst_v)

            def clampb(k, _):
                x = esrc_v[pl.ds(_mo(k * 16, 16), 16)]
                esrc_v[pl.ds(_mo(k * 16, 16), 16)] = jnp.clip(x, 0, NP - 1)
                return 0

            lax.fori_loop(0, CCH // 16, clampb, 0)
            descs = []
            for k in range(CCH // 128):
                descs.append(pltpu.async_copy(
                    hn_hbm.at[esrc_v.at[pl.ds(k * 128, 128)]],
                    rows_v.at[pl.ds(k * 128, 128)], sem))
            for dsc in descs:
                dsc.wait()
            lax.fori_loop(0, jnp.minimum(CCH, cnt - c * CCH) // 16, group_body, 0)
            return 0

        lax.fori_loop(0, nch, cbody, 0)
        return 0

    lax.fori_loop(0, 32, prod_body, 0)

    pltpu.sync_copy(acc_f, agg_hbm.at[pl.ds(_mo(t * BIN * 16, 512), BIN * 16)])


def _sc_seg_sum(bsrc, bdst, cnts, hn_pad):
    k = pl.kernel(
        _seg_sum_body,
        out_type=jax.ShapeDtypeStruct((NP * 16,), jnp.float32),
        mesh=_MESH(),
        compiler_params=_SC_PARAMS,
        scratch_types=[
            pltpu.VMEM((CCH,), jnp.int32),
            pltpu.VMEM((CCH,), jnp.int32),
            pltpu.VMEM((CCH, 16), jnp.float32),
            pltpu.VMEM((BIN * 16,), jnp.float32),
            pltpu.VMEM((NW * 32,), jnp.int32),
            pltpu.SemaphoreType.DMA,
        ],
    )
    return k(bsrc, bdst, cnts, hn_pad)


def _seg_max_body(bsrc_hbm, bdst_hbm, cnt_hbm, hpa_hbm, hpb_hbm,
                  nga_hbm, ngb_hbm,
                  esrc_v, edst_v, rowsa_v, rowsb_v, acca_f, accb_f, cnt_v,
                  sem):
    t = _wid()
    iota = _iota16()
    pltpu.sync_copy(cnt_hbm, cnt_v)

    def zacc(i, _):
        acca_f[pl.ds(_mo(i * 16, 16), 16)] = jnp.zeros((16,), jnp.float32)
        accb_f[pl.ds(_mo(i * 16, 16), 16)] = jnp.zeros((16,), jnp.float32)
        return 0

    lax.fori_loop(0, BIN, zacc, 0)

    def group_body(g, _):
        dv = edst_v[pl.ds(_mo(g * 16, 16), 16)]
        for i in range(16):
            e16 = jnp.full((16,), 0, jnp.int32) + (g * 16 + i)
            d = dv[i]
            ra = plsc.load_gather(rowsa_v, [e16, iota])
            rb = plsc.load_gather(rowsb_v, [e16, iota])
            a = acca_f[pl.ds(_mo(d * 16, 16), 16)]
            acca_f[pl.ds(_mo(d * 16, 16), 16)] = jnp.maximum(a, ra)
            b = accb_f[pl.ds(_mo(d * 16, 16), 16)]
            accb_f[pl.ds(_mo(d * 16, 16), 16)] = jnp.maximum(b, rb)
        return 0

    def prod_body(p, _):
        cnt = plsc.load_gather(
            cnt_v, [jnp.full((16,), p * 32 + t, jnp.int32)])[0]
        nch = (cnt + CCH - 1) // CCH

        def cbody(c, _):
            off = (p * 32 + t) * CAP + c * CCH
            pltpu.sync_copy(bsrc_hbm.at[pl.ds(_mo(off, 512), CCH)], esrc_v)
            pltpu.sync_copy(bdst_hbm.at[pl.ds(_mo(off, 512), CCH)], edst_v)

            def clampb(k, _):
                x = esrc_v[pl.ds(_mo(k * 16, 16), 16)]
                esrc_v[pl.ds(_mo(k * 16, 16), 16)] = jnp.clip(x, 0, NP - 1)
                return 0

            lax.fori_loop(0, CCH // 16, clampb, 0)
            descs = []
            for k in range(CCH // 128):
                descs.append(pltpu.async_copy(
                    hpa_hbm.at[esrc_v.at[pl.ds(k * 128, 128)]],
                    rowsa_v.at[pl.ds(k * 128, 128)], sem))
                descs.append(pltpu.async_copy(
                    hpb_hbm.at[esrc_v.at[pl.ds(k * 128, 128)]],
                    rowsb_v.at[pl.ds(k * 128, 128)], sem))
            for dsc in descs:
                dsc.wait()
            lax.fori_loop(0, jnp.minimum(CCH, cnt - c * CCH) // 16, group_body, 0)
            return 0

        lax.fori_loop(0, nch, cbody, 0)
        return 0

    lax.fori_loop(0, 32, prod_body, 0)

    pltpu.sync_copy(acca_f, nga_hbm.at[pl.ds(_mo(t * BIN * 16, 512), BIN * 16)])
    pltpu.sync_copy(accb_f, ngb_hbm.at[pl.ds(_mo(t * BIN * 16, 512), BIN * 16)])


def _sc_seg_max(bsrc, bdst, cnts, hpa_pad, hpb_pad):
    k = pl.kernel(
        _seg_max_body,
        out_type=[
            jax.ShapeDtypeStruct((NP * 16,), jnp.float32),
            jax.ShapeDtypeStruct((NP * 16,), jnp.float32),
        ],
        mesh=_MESH(),
        compiler_params=_SC_PARAMS,
        scratch_types=[
            pltpu.VMEM((CCH,), jnp.int32),
            pltpu.VMEM((CCH,), jnp.int32),
            pltpu.VMEM((CCH, 16), jnp.float32),
            pltpu.VMEM((CCH, 16), jnp.float32),
            pltpu.VMEM((BIN * 16,), jnp.float32),
            pltpu.VMEM((BIN * 16,), jnp.float32),
            pltpu.VMEM((NW * 32,), jnp.int32),
            pltpu.SemaphoreType.DMA,
        ],
    )
    return k(bsrc, bdst, cnts, hpa_pad, hpb_pad)


TBLK = 2048


def _prep_body(hist_ref, h_ref, hn_ref):
    i = pl.program_id(0)
    deg_s = hist_ref[0, :] + hist_ref[2, :]
    c_src = jax.lax.rsqrt(jnp.maximum(deg_s, 1.0))
    row = i * TBLK + jax.lax.broadcasted_iota(jnp.int32, (TBLK, 1), 0)
    hn = h_ref[...] * c_src[:, None]
    hn_ref[...] = jnp.where(row < N, hn, 0.0)


def _tc_prep(hist4, h_pad):
    return pl.pallas_call(
        _prep_body,
        grid=(NP // TBLK,),
        in_specs=[
            pl.BlockSpec((4, TBLK), lambda i: (0, i)),
            pl.BlockSpec((TBLK, 16), lambda i: (i, 0)),
        ],
        out_specs=pl.BlockSpec((TBLK, 16), lambda i: (i, 0)),
        out_shape=jax.ShapeDtypeStruct((NP, 16), jnp.float32),
    )(hist4, h_pad)


def _mid_body(hist_ref, agg_ref, w1_ref, b1_ref, wp_ref, bp_ref,
              h1_ref, hpa_ref, hpb_ref):
    i = pl.program_id(0)
    deg_d = hist_ref[1, :] + hist_ref[3, :]
    c_dst = jax.lax.rsqrt(jnp.maximum(deg_d, 1.0))
    aggc = agg_ref[...] * c_dst[:, None]
    h1 = jnp.maximum(aggc @ w1_ref[...] + b1_ref[...], 0.0)
    hp = jnp.maximum(h1 @ wp_ref[...] + bp_ref[...], 0.0)
    row = i * TBLK + jax.lax.broadcasted_iota(jnp.int32, (TBLK, 1), 0)
    hp = jnp.where(row < N, hp, 0.0)
    h1_ref[...] = h1
    hpa_ref[...] = hp[:, :16]
    hpb_ref[...] = hp[:, 16:]


def _tc_mid(hist4, agg_pad, W1, b1, Wpool, bpool):
    return pl.pallas_call(
        _mid_body,
        grid=(NP // TBLK,),
        in_specs=[
            pl.BlockSpec((4, TBLK), lambda i: (0, i)),
            pl.BlockSpec((TBLK, 16), lambda i: (i, 0)),
            pl.BlockSpec((16, 32), lambda i: (0, 0)),
            pl.BlockSpec((32,), lambda i: (0,)),
            pl.BlockSpec((32, 32), lambda i: (0, 0)),
            pl.BlockSpec((32,), lambda i: (0,)),
        ],
        out_specs=[
            pl.BlockSpec((TBLK, 32), lambda i: (i, 0)),
            pl.BlockSpec((TBLK, 16), lambda i: (i, 0)),
            pl.BlockSpec((TBLK, 16), lambda i: (i, 0)),
        ],
        out_shape=[
            jax.ShapeDtypeStruct((NP, 32), jnp.float32),
            jax.ShapeDtypeStruct((NP, 16), jnp.float32),
            jax.ShapeDtypeStruct((NP, 16), jnp.float32),
        ],
    )(hist4, agg_pad, W1, b1, Wpool, bpool)


BLK = 2000


def _tail_body(h1_ref, neigh_ref, wself_ref, wneigh_ref, bneigh_ref,
               wlin_ref, blin_ref, out_ref):
    i = pl.program_id(0)
    h1 = h1_ref[...]
    neigh = neigh_ref[...]
    h2 = jnp.maximum(
        h1 @ wself_ref[...] + neigh @ wneigh_ref[...] + bneigh_ref[...], 0.0)
    h3 = jnp.maximum(h2 @ wlin_ref[...] + blin_ref[...], 0.0)
    part = jnp.sum(h3, axis=0, keepdims=True)

    @pl.when(i == 0)
    def _():
        out_ref[...] = jnp.zeros_like(out_ref)

    out_ref[0:1, :] += part


def _dense_tail(h1, neigh, Wself, Wneigh, bneigh, Wlin, blin):
    return pl.pallas_call(
        _tail_body,
        grid=(N // BLK,),
        in_specs=[
            pl.BlockSpec((BLK, 32), lambda i: (i, 0)),
            pl.BlockSpec((BLK, 32), lambda i: (i, 0)),
            pl.BlockSpec((32, 64), lambda i: (0, 0)),
            pl.BlockSpec((32, 64), lambda i: (0, 0)),
            pl.BlockSpec((64,), lambda i: (0,)),
            pl.BlockSpec((64, 64), lambda i: (0, 0)),
            pl.BlockSpec((64,), lambda i: (0,)),
        ],
        out_specs=pl.BlockSpec((8, 64), lambda i: (0, 0)),
        out_shape=jax.ShapeDtypeStruct((8, 64), jnp.float32),
    )(h1, neigh, Wself, Wneigh, bneigh, Wlin, blin)


def kernel(node_tokens, edge_index, embed, W1, b1, Wpool, bpool, Wself,
           Wneigh, bneigh, Wlin, blin):
    tokens_pad = jnp.pad(node_tokens.astype(jnp.int32), (0, NP - N))
    edges3 = jnp.pad(edge_index.astype(jnp.int32), ((0, 0), (0, EP - E)),
                     constant_values=N).reshape(2, EP // 128, 128)
    h_pad = _sc_embed_gather(tokens_pad, embed)
    hist, bsrc, bdst, cnts = _sc_bin_degrees(edges3)
    hist4 = hist.reshape(4, NP)
    hn_pad = _tc_prep(hist4, h_pad)

    agg_pad = _sc_seg_sum(bsrc, bdst, cnts, hn_pad).reshape(NP, 16)
    h1_pad, hpa_pad, hpb_pad = _tc_mid(hist4, agg_pad, W1, b1, Wpool, bpool)

    nga, ngb = _sc_seg_max(bsrc, bdst, cnts, hpa_pad, hpb_pad)
    neigh = jnp.concatenate(
        [nga.reshape(NP, 16)[:N], ngb.reshape(NP, 16)[:N]], axis=1)

    part = _dense_tail(h1_pad[:N], neigh, Wself, Wneigh, bneigh, Wlin, blin)
    return jnp.sum(part, axis=0, keepdims=True)
